# baseline (speedup 1.0000x reference)
import functools
import math

import jax
import jax.numpy as jnp
import numpy as np
from jax import lax
from jax.experimental import pallas as pl
from jax.experimental.pallas import tpu as pltpu
from jax.experimental.pallas import tpu_sc as plsc

D_MODEL = 768
POS_LEN = 2048
_SCALE = math.sqrt(float(D_MODEL))

NC, NS = 2, 16
NW = NC * NS
LANES = 16


def _positional_encoding_np(length, depth):
    half = depth / 2
    positions = np.arange(length)[:, np.newaxis]
    depths = np.arange(half)[np.newaxis, :] / half
    angle_rates = 1 / 10000 ** depths
    angle_rads = positions * angle_rates
    return np.concatenate(
        [np.sin(angle_rads), np.cos(angle_rads)], axis=-1
    ).astype(np.float32)


_POS_NP = _positional_encoding_np(POS_LEN, D_MODEL)


@functools.partial(jax.jit, static_argnames=("n_total", "n_chunks", "chunk"))
def _sc_embed(xr, pos, table, n_total, n_chunks, chunk):
    d = table.shape[1]
    j_iters = d // LANES
    mesh = plsc.VectorSubcoreMesh(core_axis_name="c", subcore_axis_name="s")

    @functools.partial(
        pl.kernel,
        out_type=jax.ShapeDtypeStruct((n_total, d), jnp.float32),
        mesh=mesh,
        scratch_types=[
            pltpu.VMEM((n_chunks, chunk), jnp.int32),
            pltpu.VMEM((chunk, d), jnp.float32),
            pltpu.VMEM((chunk, d), jnp.float32),
            pltpu.SemaphoreType.DMA,
        ],
    )
    def k(x_hbm, pos_hbm, table_hbm, out_hbm, idx_v, rows_v, pos_v, sem):
        wid = lax.axis_index("s") * NC + lax.axis_index("c")
        base = wid * (n_chunks * chunk)
        t0 = lax.rem(base, POS_LEN)
        pltpu.sync_copy(x_hbm.at[wid], idx_v)
        for c in range(n_chunks):
            gather = pltpu.async_copy(table_hbm.at[idx_v.at[c]], rows_v, sem)
            pltpu.sync_copy(pos_hbm.at[pl.ds(t0 + c * chunk, chunk)], pos_v)
            gather.wait()

            def row_body(r, _):
                def lane_body(j, _):
                    o = pl.multiple_of(j * LANES, LANES)
                    rows_v[r, pl.ds(o, LANES)] = (
                        rows_v[r, pl.ds(o, LANES)] * _SCALE
                        + pos_v[r, pl.ds(o, LANES)]
                    )
                    return 0

                return lax.fori_loop(0, j_iters, lane_body, 0)

            lax.fori_loop(0, chunk, row_body, 0)
            pltpu.sync_copy(rows_v, out_hbm.at[pl.ds(base + c * chunk, chunk)])

    return k(xr, pos, table)


def kernel(x, table):
    b, t = x.shape
    n_total = b * t
    per_w = n_total // NW
    chunk = 64
    n_chunks = per_w // chunk
    xr = x.reshape(NW, n_chunks, chunk)
    pos = jnp.asarray(_POS_NP)
    out = _sc_embed(xr, pos, table, n_total, n_chunks, chunk)
    return out.reshape(b, t, table.shape[1])


# speedup vs baseline: 1.7963x; 1.7963x over previous
import functools
import math

import jax
import jax.numpy as jnp
import numpy as np
from jax import lax
from jax.experimental import pallas as pl
from jax.experimental.pallas import tpu as pltpu
from jax.experimental.pallas import tpu_sc as plsc

D_MODEL = 768
POS_LEN = 2048
_SCALE = math.sqrt(float(D_MODEL))

NC, NS = 2, 16
NW = NC * NS
LANES = 16


def _positional_encoding_np(length, depth):
    half = depth / 2
    positions = np.arange(length)[:, np.newaxis]
    depths = np.arange(half)[np.newaxis, :] / half
    angle_rates = 1 / 10000 ** depths
    angle_rads = positions * angle_rates
    return np.concatenate(
        [np.sin(angle_rads), np.cos(angle_rads)], axis=-1
    ).astype(np.float32)


_POS_NP = _positional_encoding_np(POS_LEN, D_MODEL)


@functools.partial(jax.jit, static_argnames=("n_total", "n_chunks", "chunk"))
def _sc_embed(xr, pos, table, n_total, n_chunks, chunk):
    d = table.shape[1]
    j_iters = d // LANES
    mesh = plsc.VectorSubcoreMesh(core_axis_name="c", subcore_axis_name="s")

    @functools.partial(
        pl.kernel,
        out_type=jax.ShapeDtypeStruct((n_total, d), jnp.float32),
        mesh=mesh,
        scratch_types=[
            pltpu.VMEM((n_chunks, chunk), jnp.int32),
            pltpu.VMEM((chunk, d), jnp.float32),
            pltpu.VMEM((chunk, d), jnp.float32),
            pltpu.SemaphoreType.DMA,
        ],
    )
    def k(x_hbm, pos_hbm, table_hbm, out_hbm, idx_v, rows_v, pos_v, sem):
        wid = lax.axis_index("s") * NC + lax.axis_index("c")
        base = wid * (n_chunks * chunk)
        t0 = lax.rem(base, POS_LEN)
        pltpu.sync_copy(x_hbm.at[wid], idx_v)
        for c in range(n_chunks):
            gather = pltpu.async_copy(table_hbm.at[idx_v.at[c]], rows_v, sem)
            pltpu.sync_copy(pos_hbm.at[pl.ds(t0 + c * chunk, chunk)], pos_v)
            gather.wait()

            def row_body(r, _):
                for j in range(j_iters):
                    o = j * LANES
                    rows_v[r, pl.ds(o, LANES)] = (
                        rows_v[r, pl.ds(o, LANES)] * _SCALE
                        + pos_v[r, pl.ds(o, LANES)]
                    )
                return 0

            lax.fori_loop(0, chunk, row_body, 0)
            pltpu.sync_copy(rows_v, out_hbm.at[pl.ds(base + c * chunk, chunk)])

    return k(xr, pos, table)


def kernel(x, table):
    b, t = x.shape
    n_total = b * t
    per_w = n_total // NW
    chunk = 64
    n_chunks = per_w // chunk
    xr = x.reshape(NW, n_chunks, chunk)
    pos = jnp.asarray(_POS_NP)
    out = _sc_embed(xr, pos, table, n_total, n_chunks, chunk)
    return out.reshape(b, t, table.shape[1])


# speedup vs baseline: 1.8664x; 1.0390x over previous
import functools
import math

import jax
import jax.numpy as jnp
import numpy as np
from jax import lax
from jax.experimental import pallas as pl
from jax.experimental.pallas import tpu as pltpu
from jax.experimental.pallas import tpu_sc as plsc

D_MODEL = 768
POS_LEN = 2048
_SCALE = math.sqrt(float(D_MODEL))

NC, NS = 2, 16
NW = NC * NS
LANES = 16
SUB = 32
NBUF = 3


def _positional_encoding_np(length, depth):
    half = depth / 2
    positions = np.arange(length)[:, np.newaxis]
    depths = np.arange(half)[np.newaxis, :] / half
    angle_rates = 1 / 10000 ** depths
    angle_rads = positions * angle_rates
    return np.concatenate(
        [np.sin(angle_rads), np.cos(angle_rads)], axis=-1
    ).astype(np.float32)


_POS_NP = _positional_encoding_np(POS_LEN, D_MODEL)


@functools.partial(jax.jit, static_argnames=("b_seq", "t_seq"))
def _sc_embed(xr, pos, table, b_seq, t_seq):
    d = table.shape[1]
    t_per_w = t_seq // NW
    spw = t_per_w // SUB
    nsub = b_seq * spw
    j_iters = d // LANES
    mesh = plsc.VectorSubcoreMesh(core_axis_name="c", subcore_axis_name="s")

    @functools.partial(
        pl.kernel,
        out_type=jax.ShapeDtypeStruct((b_seq * t_seq, d), jnp.float32),
        mesh=mesh,
        scratch_types=[
            pltpu.VMEM((nsub, SUB), jnp.int32),
            pltpu.VMEM((t_per_w, d), jnp.float32),
        ]
        + [pltpu.VMEM((SUB, d), jnp.float32) for _ in range(NBUF)]
        + [
            pltpu.SemaphoreType.DMA,
            pltpu.SemaphoreType.DMA,
            pltpu.SemaphoreType.DMA,
        ],
    )
    def k(x_hbm, pos_hbm, table_hbm, out_hbm, idx_v, pos_v, *rest):
        bufs, (gsem, ssem, psem) = rest[:NBUF], rest[NBUF:]
        w = lax.axis_index("s") * NC + lax.axis_index("c")
        pltpu.sync_copy(x_hbm.at[w], idx_v)
        pcopy = pltpu.async_copy(
            pos_hbm.at[pl.ds(w * t_per_w, t_per_w)], pos_v, psem
        )
        gathers = [None] * nsub
        stores = [None] * nsub
        gathers[0] = pltpu.async_copy(table_hbm.at[idx_v.at[0]], bufs[0], gsem)
        for i in range(nsub):
            if i + 1 < nsub:
                if i + 1 >= NBUF:
                    stores[i + 1 - NBUF].wait()
                gathers[i + 1] = pltpu.async_copy(
                    table_hbm.at[idx_v.at[i + 1]], bufs[(i + 1) % NBUF], gsem
                )
            gathers[i].wait()
            if i == 0:
                pcopy.wait()
            po = (i % spw) * SUB
            buf = bufs[i % NBUF]

            def row_body(r, _):
                for j in range(j_iters):
                    o = j * LANES
                    buf[r, pl.ds(o, LANES)] = (
                        buf[r, pl.ds(o, LANES)] * _SCALE
                        + pos_v[po + r, pl.ds(o, LANES)]
                    )
                return 0

            lax.fori_loop(0, SUB, row_body, 0)
            dst = (i // spw) * t_seq + w * t_per_w + po
            stores[i] = pltpu.async_copy(
                buf, out_hbm.at[pl.ds(dst, SUB)], ssem
            )
        for i in range(max(0, nsub - NBUF), nsub):
            stores[i].wait()

    return k(xr, pos, table)


def kernel(x, table):
    b, t = x.shape
    t_per_w = t // NW
    spw = t_per_w // SUB
    xr = (
        x.reshape(b, NW, spw, SUB)
        .transpose(1, 0, 2, 3)
        .reshape(NW, b * spw, SUB)
    )
    pos = jnp.asarray(_POS_NP)
    out = _sc_embed(xr, pos, table, b, t)
    return out.reshape(b, t, table.shape[1])


# speedup vs baseline: 1.8864x; 1.0107x over previous
import functools
import math

import jax
import jax.numpy as jnp
import numpy as np
from jax import lax
from jax.experimental import pallas as pl
from jax.experimental.pallas import tpu as pltpu
from jax.experimental.pallas import tpu_sc as plsc

D_MODEL = 768
POS_LEN = 2048
_SCALE = math.sqrt(float(D_MODEL))

NC, NS = 2, 16
NW = NC * NS
LANES = 16
SUB = 32
NBUF = 3


def _positional_encoding_np(length, depth):
    half = depth / 2
    positions = np.arange(length)[:, np.newaxis]
    depths = np.arange(half)[np.newaxis, :] / half
    angle_rates = 1 / 10000 ** depths
    angle_rads = positions * angle_rates
    return np.concatenate(
        [np.sin(angle_rads), np.cos(angle_rads)], axis=-1
    ).astype(np.float32)


_POS_NP = _positional_encoding_np(POS_LEN, D_MODEL)


@functools.partial(jax.jit, static_argnames=())
def _sc_embed(x, pos, table):
    b_seq, t_seq = x.shape
    d = table.shape[1]
    t_per_w = t_seq // NW
    spw = t_per_w // SUB
    nsub = b_seq * spw
    j_iters = d // LANES
    mesh = plsc.VectorSubcoreMesh(core_axis_name="c", subcore_axis_name="s")

    @functools.partial(
        pl.kernel,
        out_type=jax.ShapeDtypeStruct((b_seq * t_seq, d), jnp.float32),
        mesh=mesh,
        scratch_types=[
            pltpu.VMEM((b_seq, t_per_w), jnp.int32),
            pltpu.VMEM((t_per_w, d), jnp.float32),
        ]
        + [pltpu.VMEM((SUB, d), jnp.float32) for _ in range(NBUF)]
        + [
            pltpu.SemaphoreType.DMA,
            pltpu.SemaphoreType.DMA,
            pltpu.SemaphoreType.DMA,
        ],
    )
    def k(x_hbm, pos_hbm, table_hbm, out_hbm, idx_v, pos_v, *rest):
        bufs, (gsem, ssem, psem) = rest[:NBUF], rest[NBUF:]
        w = lax.axis_index("s") * NC + lax.axis_index("c")
        t0 = w * t_per_w
        for b in range(b_seq):
            pltpu.sync_copy(x_hbm.at[b, pl.ds(t0, t_per_w)], idx_v.at[b])
        pcopy = pltpu.async_copy(pos_hbm.at[pl.ds(t0, t_per_w)], pos_v, psem)

        def idx_ref(i):
            return idx_v.at[i // spw, pl.ds((i % spw) * SUB, SUB)]

        gathers = [None] * nsub
        stores = [None] * nsub
        gathers[0] = pltpu.async_copy(table_hbm.at[idx_ref(0)], bufs[0], gsem)
        for i in range(nsub):
            if i + 1 < nsub:
                if i + 1 >= NBUF:
                    stores[i + 1 - NBUF].wait()
                gathers[i + 1] = pltpu.async_copy(
                    table_hbm.at[idx_ref(i + 1)], bufs[(i + 1) % NBUF], gsem
                )
            gathers[i].wait()
            if i == 0:
                pcopy.wait()
            po = (i % spw) * SUB
            buf = bufs[i % NBUF]

            def row_body(r, _):
                for j in range(j_iters):
                    o = j * LANES
                    buf[r, pl.ds(o, LANES)] = (
                        buf[r, pl.ds(o, LANES)] * _SCALE
                        + pos_v[po + r, pl.ds(o, LANES)]
                    )
                return 0

            lax.fori_loop(0, SUB, row_body, 0)
            dst = (i // spw) * t_seq + t0 + po
            stores[i] = pltpu.async_copy(
                buf, out_hbm.at[pl.ds(dst, SUB)], ssem
            )
        for i in range(max(0, nsub - NBUF), nsub):
            stores[i].wait()

    return k(x, pos, table)


def kernel(x, table):
    b, t = x.shape
    pos = jnp.asarray(_POS_NP)
    out = _sc_embed(x, pos, table)
    return out.reshape(b, t, table.shape[1])


# speedup vs baseline: 2.3380x; 1.2394x over previous
import functools
import math

import jax
import jax.numpy as jnp
import numpy as np
from jax import lax
from jax.experimental import pallas as pl
from jax.experimental.pallas import tpu as pltpu
from jax.experimental.pallas import tpu_sc as plsc

D_MODEL = 768
POS_LEN = 2048
_SCALE = math.sqrt(float(D_MODEL))

NC, NS = 2, 16
NW = NC * NS
LANES = 16
SUB = 32
NBUF = 3


def _positional_encoding_np(length, depth):
    half = depth / 2
    positions = np.arange(length)[:, np.newaxis]
    depths = np.arange(half)[np.newaxis, :] / half
    angle_rates = 1 / 10000 ** depths
    angle_rads = positions * angle_rates
    return np.concatenate(
        [np.sin(angle_rads), np.cos(angle_rads)], axis=-1
    ).astype(np.float32)


_POS_NP = _positional_encoding_np(POS_LEN, D_MODEL)


@functools.partial(jax.jit, static_argnames=())
def _sc_embed(x, pos, table):
    b_seq, t_seq = x.shape
    d = table.shape[1]
    t_per_w = t_seq // NW
    spw = t_per_w // SUB
    nsub = b_seq * spw
    j_iters = d // LANES
    mesh = plsc.VectorSubcoreMesh(core_axis_name="c", subcore_axis_name="s")

    @functools.partial(
        pl.kernel,
        out_type=jax.ShapeDtypeStruct((b_seq * t_seq, d), jnp.float32),
        mesh=mesh,
        scratch_types=[
            pltpu.VMEM((b_seq, t_per_w), jnp.int32),
            pltpu.VMEM((t_per_w, d), jnp.float32),
        ]
        + [pltpu.VMEM((SUB, d), jnp.float32) for _ in range(NBUF)]
        + [
            pltpu.SemaphoreType.DMA,
            pltpu.SemaphoreType.DMA,
            pltpu.SemaphoreType.DMA,
        ],
    )
    def k(x_hbm, pos_hbm, table_hbm, out_hbm, idx_v, pos_v, *rest):
        bufs, (gsem, ssem, psem) = rest[:NBUF], rest[NBUF:]
        w = lax.axis_index("s") * NC + lax.axis_index("c")
        t0 = w * t_per_w
        for b in range(b_seq):
            pltpu.sync_copy(x_hbm.at[b, pl.ds(t0, t_per_w)], idx_v.at[b])
        pcopy = pltpu.async_copy(pos_hbm.at[pl.ds(t0, t_per_w)], pos_v, psem)

        def idx_ref(i):
            return idx_v.at[i // spw, pl.ds((i % spw) * SUB, SUB)]

        gathers = [None] * nsub
        stores = [None] * nsub
        gathers[0] = pltpu.async_copy(table_hbm.at[idx_ref(0)], bufs[0], gsem)
        for i in range(nsub):
            if i + 1 < nsub:
                if i + 1 >= NBUF:
                    stores[i + 1 - NBUF].wait()
                gathers[i + 1] = pltpu.async_copy(
                    table_hbm.at[idx_ref(i + 1)], bufs[(i + 1) % NBUF], gsem
                )
            gathers[i].wait()
            if i == 0:
                pcopy.wait()
            po = (i % spw) * SUB
            buf = bufs[i % NBUF]

            @plsc.parallel_loop(0, SUB)
            def row_body(r):
                for j in range(j_iters):
                    o = j * LANES
                    buf[r, pl.ds(o, LANES)] = (
                        buf[r, pl.ds(o, LANES)] * _SCALE
                        + pos_v[po + r, pl.ds(o, LANES)]
                    )
            dst = (i // spw) * t_seq + t0 + po
            stores[i] = pltpu.async_copy(
                buf, out_hbm.at[pl.ds(dst, SUB)], ssem
            )
        for i in range(max(0, nsub - NBUF), nsub):
            stores[i].wait()

    return k(x, pos, table)


def kernel(x, table):
    b, t = x.shape
    pos = jnp.asarray(_POS_NP)
    out = _sc_embed(x, pos, table)
    return out.reshape(b, t, table.shape[1])
